# Initial kernel scaffold; baseline (speedup 1.0000x reference)
import jax
import jax.numpy as jnp
from jax.experimental import pallas as pl


def kernel(x, params):
    raise NotImplementedError("write your pallas kernel here")


# speedup vs baseline: 6.1456x; 6.1456x over previous
import functools

import jax
import jax.numpy as jnp
from jax import lax
from jax.experimental import pallas as pl
from jax.experimental.pallas import tpu as pltpu
from jax.experimental.pallas import tpu_sc as plsc

B = 32; N = 4096; G = 128; P = 32; D = 256; PT = 3; NH = 4; NL = 4
HD = D // NH
EPS = 1e-6


def _fps_body(x0r, x1r, x2r, cxr, cyr, czr):
    X0 = x0r[...]; X1 = x1r[...]; X2 = x2r[...]
    iota = lax.broadcasted_iota(jnp.int32, (B, N), 1)
    giota = lax.broadcasted_iota(jnp.int32, (B, G), 1)
    lx = X0[:, 0:1]; ly = X1[:, 0:1]; lz = X2[:, 0:1]
    cxs = jnp.where(giota == 0, lx, 0.0)
    cys = jnp.where(giota == 0, ly, 0.0)
    czs = jnp.where(giota == 0, lz, 0.0)
    dists0 = jnp.full((B, N), jnp.inf, jnp.float32)

    def step(k, carry):
        dists, lx, ly, lz, cxs, cys, czs = carry
        d = (X0 - lx) ** 2 + (X1 - ly) ** 2 + (X2 - lz) ** 2
        dists = jnp.minimum(dists, d)
        m = jnp.max(dists, axis=1, keepdims=True)
        nxt = jnp.min(jnp.where(dists == m, iota, N), axis=1, keepdims=True)
        oh = (iota == nxt).astype(jnp.float32)
        lx = jnp.sum(X0 * oh, axis=1, keepdims=True)
        ly = jnp.sum(X1 * oh, axis=1, keepdims=True)
        lz = jnp.sum(X2 * oh, axis=1, keepdims=True)
        cxs = jnp.where(giota == k, lx, cxs)
        cys = jnp.where(giota == k, ly, cys)
        czs = jnp.where(giota == k, lz, czs)
        return dists, lx, ly, lz, cxs, cys, czs

    (_, _, _, _, cxs, cys, czs) = lax.fori_loop(
        1, G, step, (dists0, lx, ly, lz, cxs, cys, czs))
    cxr[...] = cxs; cyr[...] = cys; czr[...] = czs


def _fps(x0, x1, x2):
    out = [jax.ShapeDtypeStruct((B, G), jnp.float32)] * 3
    return pl.pallas_call(_fps_body, out_shape=out)(x0, x1, x2)


def _knn_body(x8r, cpr, idxr):
    b = pl.program_id(0)
    x8 = x8r[0]
    cp = cpr[0]
    xn2 = ((x8[0:1] * x8[0:1] + x8[1:2] * x8[1:2])
           + x8[2:3] * x8[2:3])
    cn2 = jnp.sum(cp * cp, axis=1, keepdims=True)
    dot = lax.dot_general(cp.astype(jnp.bfloat16), x8.astype(jnp.bfloat16),
                          (((1,), (0,)), ((), ())),
                          preferred_element_type=jnp.float32)
    d = (cn2 + xn2) - 2.0 * dot
    iota = lax.broadcasted_iota(jnp.int32, (G, N), 1)
    cols = []
    for _ in range(P):
        m = jnp.min(d, axis=1, keepdims=True)
        nj = jnp.min(jnp.where(d == m, iota, N), axis=1, keepdims=True)
        cols.append(nj)
        d = jnp.where(iota == nj, jnp.float32(jnp.inf), d)
    idx = jnp.concatenate(cols, axis=1)
    idxr[0] = idx + b * N


def _knn(x8, cpad):
    return pl.pallas_call(
        _knn_body,
        grid=(B,),
        in_specs=[
            pl.BlockSpec((1, 8, N), lambda b: (b, 0, 0)),
            pl.BlockSpec((1, G, 8), lambda b: (b, 0, 0)),
        ],
        out_specs=pl.BlockSpec((1, G, P), lambda b: (b, 0, 0)),
        out_shape=jax.ShapeDtypeStruct((B, G, P), jnp.int32),
    )(x8, cpad)


def _gather(x0f, x1f, x2f, idx):
    info = plsc.get_sparse_core_info()
    nw = info.num_cores * info.num_subcores
    n_idx = B * G * P
    per_w = n_idx // nw
    chunk = 128
    n_chunks = per_w // chunk
    mesh = plsc.VectorSubcoreMesh(core_axis_name="c", subcore_axis_name="s")

    @functools.partial(
        pl.kernel, mesh=mesh,
        compiler_params=pltpu.CompilerParams(use_tc_tiling_on_sc=False),
        out_type=[jax.ShapeDtypeStruct((n_idx,), jnp.float32)] * 3,
        scratch_types=[
            pltpu.VMEM((per_w,), jnp.int32),
            pltpu.VMEM((per_w,), jnp.float32),
            pltpu.VMEM((per_w,), jnp.float32),
            pltpu.VMEM((per_w,), jnp.float32),
            pltpu.SemaphoreType.DMA,
        ],
    )
    def k(x0_hbm, x1_hbm, x2_hbm, idx_hbm, o0_hbm, o1_hbm, o2_hbm,
          idx_v, v0, v1, v2, sem):
        wid = lax.axis_index("s") * info.num_cores + lax.axis_index("c")
        base = wid * per_w
        pltpu.sync_copy(idx_hbm.at[pl.ds(base, per_w)], idx_v)
        copies = []
        for src, dst in ((x0_hbm, v0), (x1_hbm, v1), (x2_hbm, v2)):
            for j in range(n_chunks):
                copies.append(pltpu.async_copy(
                    src.at[idx_v.at[pl.ds(j * chunk, chunk)]],
                    dst.at[pl.ds(j * chunk, chunk)], sem))
        for c in copies:
            c.wait()
        for dst, out in ((v0, o0_hbm), (v1, o1_hbm), (v2, o2_hbm)):
            pltpu.sync_copy(dst, out.at[pl.ds(base, per_w)])

    return k(x0f, x1f, x2f, idx)


def _ln(x, g, b):
    m = jnp.mean(x, -1, keepdims=True)
    v = jnp.mean((x - m) ** 2, -1, keepdims=True)
    return (x - m) / jnp.sqrt(v + EPS) * g + b


def _tokens_body(*refs):
    (pr, cr, w1r, b1r, g1r, be1r, w2r, b2r, g2r, be2r, w3r, b3r,
     wp1r, bp1r, wp2r, bp2r) = refs[:16]
    blocks = refs[16:16 + 12 * NL]
    outr = refs[16 + 12 * NL]

    p = pr[0]
    c = cr[0]
    lane = lax.broadcasted_iota(jnp.int32, (G, P * PT), 1) % PT
    csub = (jnp.where(lane == 0, c[:, 0:1], 0.0)
            + jnp.where(lane == 1, c[:, 1:2], 0.0)
            + jnp.where(lane == 2, c[:, 2:3], 0.0))
    pc = p - csub

    def mm(a, w):
        return lax.dot_general(a, w, (((1,), (0,)), ((), ())),
                               preferred_element_type=jnp.float32)

    h = mm(pc, w1r[...]) + b1r[...]
    h = jax.nn.gelu(_ln(h, g1r[...], be1r[...]))
    h = mm(h, w2r[...]) + b2r[...]
    h = jax.nn.gelu(_ln(h, g2r[...], be2r[...]))
    z = mm(h, w3r[...]) + b3r[...]

    pe = jax.nn.gelu(mm(c, wp1r[...]) + bp1r[...])
    pe = mm(pe, wp2r[...]) + bp2r[...]
    z = z + pe

    for l in range(NL):
        (l1g, l1b, wqkv, bqkv, wproj, bproj,
         l2g, l2b, wf1, bf1, wf2, bf2) = blocks[12 * l:12 * (l + 1)]
        y = _ln(z, l1g[...], l1b[...])
        qkv = mm(y, wqkv[...]) + bqkv[...]
        outs = []
        for hh in range(NH):
            q = qkv[:, hh * HD:(hh + 1) * HD]
            k_ = qkv[:, D + hh * HD:D + (hh + 1) * HD]
            v = qkv[:, 2 * D + hh * HD:2 * D + (hh + 1) * HD]
            s = lax.dot_general(q, k_, (((1,), (1,)), ((), ())),
                                preferred_element_type=jnp.float32)
            s = s * (HD ** -0.5)
            s = s - jnp.max(s, axis=-1, keepdims=True)
            e = jnp.exp(s)
            a = e / jnp.sum(e, axis=-1, keepdims=True)
            outs.append(mm(a, v))
        o = jnp.concatenate(outs, axis=1)
        z = z + mm(o, wproj[...]) + bproj[...]
        y2 = _ln(z, l2g[...], l2b[...])
        hmlp = jax.nn.gelu(mm(y2, wf1[...]) + bf1[...])
        z = z + mm(hmlp, wf2[...]) + bf2[...]
    outr[0] = z


def _tokens(p_tok, cpad, weights):
    ins = [p_tok, cpad] + weights
    specs = [
        pl.BlockSpec((1, G, P * PT), lambda b: (b, 0, 0)),
        pl.BlockSpec((1, G, 8), lambda b: (b, 0, 0)),
    ]
    for w in weights:
        nd = w.ndim
        specs.append(pl.BlockSpec(w.shape, lambda b, _nd=nd: (0,) * _nd))
    return pl.pallas_call(
        _tokens_body,
        grid=(B,),
        in_specs=specs,
        out_specs=pl.BlockSpec((1, G, D), lambda b: (b, 0, 0)),
        out_shape=jax.ShapeDtypeStruct((B, G, D), jnp.float32),
    )(*ins)


def kernel(x, params):
    xt = jnp.transpose(x, (0, 2, 1))
    x0, x1, x2 = xt[:, 0], xt[:, 1], xt[:, 2]
    x8 = jnp.pad(xt, ((0, 0), (0, 5), (0, 0)))

    cx, cy, cz = _fps(x0, x1, x2)
    cpad = jnp.pad(jnp.stack([cx, cy, cz], axis=-1),
                   ((0, 0), (0, 0), (0, 5)))

    gidx = _knn(x8, cpad)
    o0, o1, o2 = _gather(x0.reshape(-1), x1.reshape(-1), x2.reshape(-1),
                         gidx.reshape(-1))
    p_tok = jnp.stack([o0, o1, o2], axis=-1).reshape(B, G, P * PT)

    pa = params["patch"]
    po = params["pos"]
    wp1p = jnp.pad(po[0]["W"], ((0, 5), (0, 0)))
    row = lambda v: v.reshape(1, -1)
    weights = [pa[0]["W"], row(pa[0]["b"]), row(pa[0]["g"]), row(pa[0]["beta"]),
               pa[1]["W"], row(pa[1]["b"]), row(pa[1]["g"]), row(pa[1]["beta"]),
               pa[2]["W"], row(pa[2]["b"]),
               wp1p, row(po[0]["b"]), po[1]["W"], row(po[1]["b"])]
    for blk in params["blocks"]:
        weights += [row(blk["ln1_g"]), row(blk["ln1_b"]),
                    blk["qkv"]["W"], row(blk["qkv"]["b"]),
                    blk["proj"]["W"], row(blk["proj"]["b"]),
                    row(blk["ln2_g"]), row(blk["ln2_b"]),
                    blk["fc1"]["W"], row(blk["fc1"]["b"]),
                    blk["fc2"]["W"], row(blk["fc2"]["b"])]
    return _tokens(p_tok, cpad, weights)


# speedup vs baseline: 6.9780x; 1.1354x over previous
import functools

import jax
import jax.numpy as jnp
from jax import lax
from jax.experimental import pallas as pl
from jax.experimental.pallas import tpu as pltpu
from jax.experimental.pallas import tpu_sc as plsc

B = 32; N = 4096; G = 128; P = 32; D = 256; PT = 3; NH = 4; NL = 4
HD = D // NH
EPS = 1e-6


def _fps_body(x0r, x1r, x2r, cxr, cyr, czr):
    X0 = x0r[...]; X1 = x1r[...]; X2 = x2r[...]
    iota = lax.broadcasted_iota(jnp.int32, (B, N), 1)
    giota = lax.broadcasted_iota(jnp.int32, (B, G), 1)
    lx = X0[:, 0:1]; ly = X1[:, 0:1]; lz = X2[:, 0:1]
    cxs = jnp.where(giota == 0, lx, 0.0)
    cys = jnp.where(giota == 0, ly, 0.0)
    czs = jnp.where(giota == 0, lz, 0.0)
    dists0 = jnp.full((B, N), jnp.inf, jnp.float32)

    def step(k, carry):
        dists, lx, ly, lz, cxs, cys, czs = carry
        d = (X0 - lx) ** 2 + (X1 - ly) ** 2 + (X2 - lz) ** 2
        dists = jnp.minimum(dists, d)
        m = jnp.max(dists, axis=1, keepdims=True)
        nxt = jnp.min(jnp.where(dists == m, iota, N), axis=1, keepdims=True)
        oh = (iota == nxt).astype(jnp.float32)
        lx = jnp.sum(X0 * oh, axis=1, keepdims=True)
        ly = jnp.sum(X1 * oh, axis=1, keepdims=True)
        lz = jnp.sum(X2 * oh, axis=1, keepdims=True)
        cxs = jnp.where(giota == k, lx, cxs)
        cys = jnp.where(giota == k, ly, cys)
        czs = jnp.where(giota == k, lz, czs)
        return dists, lx, ly, lz, cxs, cys, czs

    (_, _, _, _, cxs, cys, czs) = lax.fori_loop(
        1, G, step, (dists0, lx, ly, lz, cxs, cys, czs))
    cxr[...] = cxs; cyr[...] = cys; czr[...] = czs


def _fps(x0, x1, x2):
    out = [jax.ShapeDtypeStruct((B, G), jnp.float32)] * 3
    return pl.pallas_call(_fps_body, out_shape=out)(x0, x1, x2)


def _knn_body(x8r, cpr, idxr):
    b = pl.program_id(0)
    x8 = x8r[0]
    cp = cpr[0]
    xn2 = ((x8[0:1] * x8[0:1] + x8[1:2] * x8[1:2])
           + x8[2:3] * x8[2:3])
    cn2 = jnp.sum(cp * cp, axis=1, keepdims=True)
    dot = lax.dot_general(cp.astype(jnp.bfloat16), x8.astype(jnp.bfloat16),
                          (((1,), (0,)), ((), ())),
                          preferred_element_type=jnp.float32)
    d = (cn2 + xn2) - 2.0 * dot
    iota = lax.broadcasted_iota(jnp.int32, (G, N), 1)
    cols = []
    for _ in range(P):
        m = jnp.min(d, axis=1, keepdims=True)
        nj = jnp.min(jnp.where(d == m, iota, N), axis=1, keepdims=True)
        cols.append(nj)
        d = jnp.where(iota == nj, jnp.float32(jnp.inf), d)
    idx = jnp.concatenate(cols, axis=1)
    idxr[0] = idx + b * N


def _knn(x8, cpad):
    return pl.pallas_call(
        _knn_body,
        grid=(B,),
        in_specs=[
            pl.BlockSpec((1, 8, N), lambda b: (b, 0, 0)),
            pl.BlockSpec((1, G, 8), lambda b: (b, 0, 0)),
        ],
        out_specs=pl.BlockSpec((1, G, P), lambda b: (b, 0, 0)),
        out_shape=jax.ShapeDtypeStruct((B, G, P), jnp.int32),
    )(x8, cpad)


def _gather(x0f, x1f, x2f, idx):
    info = plsc.get_sparse_core_info()
    nw = info.num_cores * info.num_subcores
    n_idx = B * G * P
    per_w = n_idx // nw
    chunk = 128
    n_chunks = per_w // chunk
    mesh = plsc.VectorSubcoreMesh(core_axis_name="c", subcore_axis_name="s")

    @functools.partial(
        pl.kernel, mesh=mesh,
        compiler_params=pltpu.CompilerParams(use_tc_tiling_on_sc=False),
        out_type=[jax.ShapeDtypeStruct((n_idx,), jnp.float32)] * 3,
        scratch_types=[
            pltpu.VMEM((per_w,), jnp.int32),
            pltpu.VMEM((per_w,), jnp.float32),
            pltpu.VMEM((per_w,), jnp.float32),
            pltpu.VMEM((per_w,), jnp.float32),
            pltpu.SemaphoreType.DMA,
        ],
    )
    def k(x0_hbm, x1_hbm, x2_hbm, idx_hbm, o0_hbm, o1_hbm, o2_hbm,
          idx_v, v0, v1, v2, sem):
        wid = lax.axis_index("s") * info.num_cores + lax.axis_index("c")
        base = wid * per_w
        pltpu.sync_copy(idx_hbm.at[pl.ds(base, per_w)], idx_v)
        copies = []
        for src, dst in ((x0_hbm, v0), (x1_hbm, v1), (x2_hbm, v2)):
            for j in range(n_chunks):
                copies.append(pltpu.async_copy(
                    src.at[idx_v.at[pl.ds(j * chunk, chunk)]],
                    dst.at[pl.ds(j * chunk, chunk)], sem))
        for c in copies:
            c.wait()
        for dst, out in ((v0, o0_hbm), (v1, o1_hbm), (v2, o2_hbm)):
            pltpu.sync_copy(dst, out.at[pl.ds(base, per_w)])

    return k(x0f, x1f, x2f, idx)


def _ln(x, g, b):
    m = jnp.mean(x, -1, keepdims=True)
    v = jnp.mean((x - m) ** 2, -1, keepdims=True)
    return (x - m) / jnp.sqrt(v + EPS) * g + b


CB = 8
T = CB * G


def _tokens_body(*refs):
    (pr, cr, w1r, b1r, g1r, be1r, w2r, b2r, g2r, be2r, w3r, b3r,
     wp1r, bp1r, wp2r, bp2r) = refs[:16]
    blocks = refs[16:16 + 12 * NL]
    outr = refs[16 + 12 * NL]

    p = pr[...].reshape(T, P * PT)
    c = cr[...].reshape(T, 8)
    lane = lax.broadcasted_iota(jnp.int32, (T, P * PT), 1) % PT
    csub = (jnp.where(lane == 0, c[:, 0:1], 0.0)
            + jnp.where(lane == 1, c[:, 1:2], 0.0)
            + jnp.where(lane == 2, c[:, 2:3], 0.0))
    pc = p - csub

    def mm(a, w):
        return lax.dot_general(a, w, (((1,), (0,)), ((), ())),
                               preferred_element_type=jnp.float32)

    h = mm(pc, w1r[...]) + b1r[...]
    h = jax.nn.gelu(_ln(h, g1r[...], be1r[...]))
    h = mm(h, w2r[...]) + b2r[...]
    h = jax.nn.gelu(_ln(h, g2r[...], be2r[...]))
    z = mm(h, w3r[...]) + b3r[...]

    pe = jax.nn.gelu(mm(c, wp1r[...]) + bp1r[...])
    pe = mm(pe, wp2r[...]) + bp2r[...]
    z = z + pe

    ri = lax.broadcasted_iota(jnp.int32, (T, T), 0) // G
    ci = lax.broadcasted_iota(jnp.int32, (T, T), 1) // G
    blkmask = ri == ci
    NEG = jnp.float32(-1e30)

    for l in range(NL):
        (l1g, l1b, wqkv, bqkv, wproj, bproj,
         l2g, l2b, wf1, bf1, wf2, bf2) = blocks[12 * l:12 * (l + 1)]
        y = _ln(z, l1g[...], l1b[...])
        qkv = mm(y, wqkv[...]) + bqkv[...]
        outs = []
        for hh in range(NH):
            q = qkv[:, hh * HD:(hh + 1) * HD]
            k_ = qkv[:, D + hh * HD:D + (hh + 1) * HD]
            v = qkv[:, 2 * D + hh * HD:2 * D + (hh + 1) * HD]
            s = lax.dot_general(q, k_, (((1,), (1,)), ((), ())),
                                preferred_element_type=jnp.float32)
            s = jnp.where(blkmask, s * (HD ** -0.5), NEG)
            s = s - jnp.max(s, axis=-1, keepdims=True)
            e = jnp.exp(s)
            a = e / jnp.sum(e, axis=-1, keepdims=True)
            outs.append(mm(a, v))
        o = jnp.concatenate(outs, axis=1)
        z = z + mm(o, wproj[...]) + bproj[...]
        y2 = _ln(z, l2g[...], l2b[...])
        hmlp = jax.nn.gelu(mm(y2, wf1[...]) + bf1[...])
        z = z + mm(hmlp, wf2[...]) + bf2[...]
    outr[...] = z.reshape(CB, G, D)


def _tokens(p_tok, cpad, weights):
    ins = [p_tok, cpad] + weights
    specs = [
        pl.BlockSpec((CB, G, P * PT), lambda b: (b, 0, 0)),
        pl.BlockSpec((CB, G, 8), lambda b: (b, 0, 0)),
    ]
    for w in weights:
        nd = w.ndim
        specs.append(pl.BlockSpec(w.shape, lambda b, _nd=nd: (0,) * _nd))
    return pl.pallas_call(
        _tokens_body,
        grid=(B // CB,),
        in_specs=specs,
        out_specs=pl.BlockSpec((CB, G, D), lambda b: (b, 0, 0)),
        out_shape=jax.ShapeDtypeStruct((B, G, D), jnp.float32),
    )(*ins)


def kernel(x, params):
    xt = jnp.transpose(x, (0, 2, 1))
    x0, x1, x2 = xt[:, 0], xt[:, 1], xt[:, 2]
    x8 = jnp.pad(xt, ((0, 0), (0, 5), (0, 0)))

    cx, cy, cz = _fps(x0, x1, x2)
    cpad = jnp.pad(jnp.stack([cx, cy, cz], axis=-1),
                   ((0, 0), (0, 0), (0, 5)))

    gidx = _knn(x8, cpad)
    o0, o1, o2 = _gather(x0.reshape(-1), x1.reshape(-1), x2.reshape(-1),
                         gidx.reshape(-1))
    p_tok = jnp.stack([o0, o1, o2], axis=-1).reshape(B, G, P * PT)

    pa = params["patch"]
    po = params["pos"]
    wp1p = jnp.pad(po[0]["W"], ((0, 5), (0, 0)))
    row = lambda v: v.reshape(1, -1)
    weights = [pa[0]["W"], row(pa[0]["b"]), row(pa[0]["g"]), row(pa[0]["beta"]),
               pa[1]["W"], row(pa[1]["b"]), row(pa[1]["g"]), row(pa[1]["beta"]),
               pa[2]["W"], row(pa[2]["b"]),
               wp1p, row(po[0]["b"]), po[1]["W"], row(po[1]["b"])]
    for blk in params["blocks"]:
        weights += [row(blk["ln1_g"]), row(blk["ln1_b"]),
                    blk["qkv"]["W"], row(blk["qkv"]["b"]),
                    blk["proj"]["W"], row(blk["proj"]["b"]),
                    row(blk["ln2_g"]), row(blk["ln2_b"]),
                    blk["fc1"]["W"], row(blk["fc1"]["b"]),
                    blk["fc2"]["W"], row(blk["fc2"]["b"])]
    return _tokens(p_tok, cpad, weights)
